# Initial kernel scaffold; baseline (speedup 1.0000x reference)
import jax
import jax.numpy as jnp
from jax.experimental import pallas as pl


def kernel(x, edge_index, adj_values, W1, b1, W2, b2, W3, b3):
    raise NotImplementedError("write your pallas kernel here")


# speedup vs baseline: 3.3500x; 3.3500x over previous
import functools

import jax
import jax.numpy as jnp
from jax import lax
from jax.experimental import pallas as pl
from jax.experimental.pallas import tpu as pltpu
from jax.experimental.pallas import tpu_sc as plsc

N = 10000
E = 320000
D = 128

NC = 2
NS = 16
NW = NC * NS
EPW = E // NW
C = 80
NCH = EPW // C
RPT = 624
ZR = 16

RB = 1000


def _sc_spmv_body(sup_hbm, src_hbm, dst_hbm, val_hbm, out_hbm,
                  acc_sh, zbuf, idx_s, idx_d, valv, rows, sem):
    c = lax.axis_index("c")
    s = lax.axis_index("s")
    wid = c * NS + s

    zeros16 = jnp.zeros((16,), jnp.float32)
    for i in range(ZR):
        for j in range(D // 16):
            zbuf[i, pl.ds(j * 16, 16)] = zeros16

    def zero_body(k, carry):
        pltpu.sync_copy(zbuf, acc_sh.at[pl.ds(s * RPT + k * ZR, ZR)])
        return carry

    nz = RPT // ZR + jnp.where(s == NS - 1, (N - NS * RPT) // ZR, 0)
    lax.fori_loop(0, nz, zero_body, 0)
    plsc.subcore_barrier()

    base = wid * EPW

    def chunk_body(t, carry):
        off = base + t * C
        pltpu.sync_copy(src_hbm.at[pl.ds(off, C)], idx_s)
        pltpu.sync_copy(dst_hbm.at[pl.ds(off, C)], idx_d)
        pltpu.sync_copy(val_hbm.at[pl.ds(off, C)], valv)
        pltpu.async_copy(sup_hbm.at[idx_s], rows, sem).wait()

        def edge_body(e, carry2):
            bc = plsc.load_gather(valv, [jnp.zeros((16,), jnp.int32) + e])
            for j in range(D // 16):
                rows[e, pl.ds(j * 16, 16)] = rows[e, pl.ds(j * 16, 16)] * bc
            return carry2

        lax.fori_loop(0, C, edge_body, 0)
        pltpu.sync_copy(rows, acc_sh.at[idx_d], add=True)
        return carry

    lax.fori_loop(0, NCH, chunk_body, 0)

    plsc.subcore_barrier()
    pltpu.sync_copy(acc_sh.at[pl.ds(s * RPT, RPT)],
                    out_hbm.at[c, pl.ds(s * RPT, RPT)])

    @pl.when(s == NS - 1)
    def _tail():
        pltpu.sync_copy(acc_sh.at[pl.ds(NS * RPT, N - NS * RPT)],
                        out_hbm.at[c, pl.ds(NS * RPT, N - NS * RPT)])


_sc_spmv = pl.kernel(
    _sc_spmv_body,
    out_type=jax.ShapeDtypeStruct((NC, N, D), jnp.float32),
    mesh=plsc.VectorSubcoreMesh(core_axis_name="c", subcore_axis_name="s",
                                num_cores=NC, num_subcores=NS),
    scratch_types=[
        pltpu.MemorySpace.VMEM_SHARED((N, D), jnp.float32),
        pltpu.VMEM((ZR, D), jnp.float32),
        pltpu.VMEM((C,), jnp.int32),
        pltpu.VMEM((C,), jnp.int32),
        pltpu.VMEM((C,), jnp.float32),
        pltpu.VMEM((C, D), jnp.float32),
        pltpu.SemaphoreType.DMA,
    ],
    compiler_params=pltpu.CompilerParams(needs_layout_passes=False),
)


def _mm_plain_body(x_ref, w_ref, o_ref):
    o_ref[...] = jnp.dot(x_ref[...], w_ref[...],
                         preferred_element_type=jnp.float32)


def _mm_plain(x, W):
    return pl.pallas_call(
        _mm_plain_body,
        grid=(N // RB,),
        in_specs=[pl.BlockSpec((RB, D), lambda i: (i, 0)),
                  pl.BlockSpec((D, D), lambda i: (0, 0))],
        out_specs=pl.BlockSpec((RB, D), lambda i: (i, 0)),
        out_shape=jax.ShapeDtypeStruct((N, D), jnp.float32),
    )(x, W)


def _mm_fused_body(p_ref, b_ref, w_ref, o_ref):
    h = jnp.maximum(p_ref[0] + p_ref[1] + b_ref[...], 0.0)
    o_ref[...] = jnp.dot(h, w_ref[...], preferred_element_type=jnp.float32)


def _mm_fused(p, b, W):
    return pl.pallas_call(
        _mm_fused_body,
        grid=(N // RB,),
        in_specs=[pl.BlockSpec((NC, RB, D), lambda i: (0, i, 0)),
                  pl.BlockSpec((1, D), lambda i: (0, 0)),
                  pl.BlockSpec((D, D), lambda i: (0, 0))],
        out_specs=pl.BlockSpec((RB, D), lambda i: (i, 0)),
        out_shape=jax.ShapeDtypeStruct((N, D), jnp.float32),
    )(p, b.reshape(1, D), W)


def _final_body(p_ref, b_ref, o_ref):
    o_ref[...] = p_ref[0] + p_ref[1] + b_ref[...]


def _final(p, b):
    return pl.pallas_call(
        _final_body,
        grid=(N // RB,),
        in_specs=[pl.BlockSpec((NC, RB, D), lambda i: (0, i, 0)),
                  pl.BlockSpec((1, D), lambda i: (0, 0))],
        out_specs=pl.BlockSpec((RB, D), lambda i: (i, 0)),
        out_shape=jax.ShapeDtypeStruct((N, D), jnp.float32),
    )(p, b.reshape(1, D))


def kernel(x, edge_index, adj_values, W1, b1, W2, b2, W3, b3):
    dst = edge_index[0]
    src = edge_index[1]

    sup1 = _mm_plain(x, W1)
    p1 = _sc_spmv(sup1, src, dst, adj_values)
    sup2 = _mm_fused(p1, b1, W2)
    p2 = _sc_spmv(sup2, src, dst, adj_values)
    sup3 = _mm_fused(p2, b2, W3)
    p3 = _sc_spmv(sup3, src, dst, adj_values)
    return _final(p3, b3)


# speedup vs baseline: 9.9566x; 2.9721x over previous
import functools

import jax
import jax.numpy as jnp
from jax import lax
from jax.experimental import pallas as pl
from jax.experimental.pallas import tpu as pltpu
from jax.experimental.pallas import tpu_sc as plsc

N = 10000
E = 320000
D = 128

NC = 2
NS = 16
NW = NC * NS
EPW = E // NW
C = 80
NCH = EPW // C
RPT = 624
ZR = 16

RB = 1000


NSLOT = 4
UNR = 4


def _sc_spmv_body(sup_hbm, src_hbm, dst_hbm, val_hbm, out_hbm,
                  acc_sh, zbuf,
                  is0, is1, is2, is3, id0, id1, id2, id3,
                  vv0, vv1, vv2, vv3, r0, r1, r2, r3,
                  si0, si1, si2, si3, sg0, sg1, sg2, sg3,
                  ss0, ss1, ss2, ss3):
    IS = [is0, is1, is2, is3]
    ID = [id0, id1, id2, id3]
    VV = [vv0, vv1, vv2, vv3]
    RW = [r0, r1, r2, r3]
    SI = [si0, si1, si2, si3]
    SG = [sg0, sg1, sg2, sg3]
    SS = [ss0, ss1, ss2, ss3]

    c = lax.axis_index("c")
    s = lax.axis_index("s")
    wid = c * NS + s

    zeros16 = jnp.zeros((16,), jnp.float32)
    for i in range(ZR):
        for j in range(D // 16):
            zbuf[i, pl.ds(j * 16, 16)] = zeros16

    def zero_body(k, carry):
        pltpu.sync_copy(zbuf, acc_sh.at[pl.ds(s * RPT + k * ZR, ZR)])
        return carry

    nz = RPT // ZR + jnp.where(s == NS - 1, (N - NS * RPT) // ZR, 0)
    lax.fori_loop(0, nz, zero_body, 0)
    plsc.subcore_barrier()

    base = wid * EPW

    def idx_start(t, b):
        off = base + t * C
        pltpu.async_copy(src_hbm.at[pl.ds(off, C)], IS[b], SI[b])
        pltpu.async_copy(dst_hbm.at[pl.ds(off, C)], ID[b], SI[b])
        pltpu.async_copy(val_hbm.at[pl.ds(off, C)], VV[b], SI[b])

    def idx_wait(b):
        pltpu.make_async_copy(src_hbm.at[pl.ds(0, C)], IS[b], SI[b]).wait()
        pltpu.make_async_copy(dst_hbm.at[pl.ds(0, C)], ID[b], SI[b]).wait()
        pltpu.make_async_copy(val_hbm.at[pl.ds(0, C)], VV[b], SI[b]).wait()

    def gather_start(b):
        pltpu.async_copy(sup_hbm.at[IS[b]], RW[b], SG[b])

    def gather_wait(b):
        pltpu.make_async_copy(sup_hbm.at[IS[b]], RW[b], SG[b]).wait()

    def scatter_start(b):
        pltpu.async_copy(RW[b], acc_sh.at[ID[b]], SS[b], add=True)

    def scatter_wait(b):
        pltpu.make_async_copy(RW[b], acc_sh.at[ID[b]], SS[b]).wait()

    def scale(b):
        def ebody(e, carry2):
            for u in range(UNR):
                ee = e * UNR + u
                bc = plsc.load_gather(VV[b], [jnp.zeros((16,), jnp.int32) + ee])
                for j in range(D // 16):
                    RW[b][ee, pl.ds(j * 16, 16)] = (
                        RW[b][ee, pl.ds(j * 16, 16)] * bc)
            return carry2

        lax.fori_loop(0, C // UNR, ebody, 0)

    def body(t, b, wait_scatter=True, do_idx=True, do_gather=True):
        if wait_scatter:
            scatter_wait((b + 2) % NSLOT)
        if do_idx:
            idx_start(t + 2, (b + 2) % NSLOT)
        if do_gather:
            idx_wait((b + 1) % NSLOT)
            gather_start((b + 1) % NSLOT)
        gather_wait(b)
        scale(b)
        scatter_start(b)

    idx_start(0, 0)
    idx_start(1, 1)
    idx_wait(0)
    gather_start(0)
    body(0, 0, wait_scatter=False)
    body(1, 1, wait_scatter=False)

    def quad(p, carry):
        t0 = 2 + p * NSLOT
        for u in range(NSLOT):
            body(t0 + u, (2 + u) % NSLOT)
        return carry

    lax.fori_loop(0, (NCH - 5) // NSLOT, quad, 0)

    body(NCH - 3, (NCH - 3) % NSLOT)
    body(NCH - 2, (NCH - 2) % NSLOT, do_idx=False, do_gather=True)
    body(NCH - 1, (NCH - 1) % NSLOT, do_idx=False, do_gather=False)
    scatter_wait((NCH - 2) % NSLOT)
    scatter_wait((NCH - 1) % NSLOT)

    plsc.subcore_barrier()
    pltpu.sync_copy(acc_sh.at[pl.ds(s * RPT, RPT)],
                    out_hbm.at[c, pl.ds(s * RPT, RPT)])

    @pl.when(s == NS - 1)
    def _tail():
        pltpu.sync_copy(acc_sh.at[pl.ds(NS * RPT, N - NS * RPT)],
                        out_hbm.at[c, pl.ds(NS * RPT, N - NS * RPT)])


_sc_spmv = pl.kernel(
    _sc_spmv_body,
    out_type=jax.ShapeDtypeStruct((NC, N, D), jnp.float32),
    mesh=plsc.VectorSubcoreMesh(core_axis_name="c", subcore_axis_name="s",
                                num_cores=NC, num_subcores=NS),
    scratch_types=(
        [pltpu.MemorySpace.VMEM_SHARED((N, D), jnp.float32),
         pltpu.VMEM((ZR, D), jnp.float32)]
        + [pltpu.VMEM((C,), jnp.int32) for _ in range(2 * NSLOT)]
        + [pltpu.VMEM((C,), jnp.float32) for _ in range(NSLOT)]
        + [pltpu.VMEM((C, D), jnp.float32) for _ in range(NSLOT)]
        + [pltpu.SemaphoreType.DMA for _ in range(3 * NSLOT)]
    ),
    compiler_params=pltpu.CompilerParams(needs_layout_passes=False),
)


def _mm_plain_body(x_ref, w_ref, o_ref):
    o_ref[...] = jnp.dot(x_ref[...], w_ref[...],
                         preferred_element_type=jnp.float32)


def _mm_plain(x, W):
    return pl.pallas_call(
        _mm_plain_body,
        grid=(N // RB,),
        in_specs=[pl.BlockSpec((RB, D), lambda i: (i, 0)),
                  pl.BlockSpec((D, D), lambda i: (0, 0))],
        out_specs=pl.BlockSpec((RB, D), lambda i: (i, 0)),
        out_shape=jax.ShapeDtypeStruct((N, D), jnp.float32),
    )(x, W)


def _mm_fused_body(p_ref, b_ref, w_ref, o_ref):
    h = jnp.maximum(p_ref[0] + p_ref[1] + b_ref[...], 0.0)
    o_ref[...] = jnp.dot(h, w_ref[...], preferred_element_type=jnp.float32)


def _mm_fused(p, b, W):
    return pl.pallas_call(
        _mm_fused_body,
        grid=(N // RB,),
        in_specs=[pl.BlockSpec((NC, RB, D), lambda i: (0, i, 0)),
                  pl.BlockSpec((1, D), lambda i: (0, 0)),
                  pl.BlockSpec((D, D), lambda i: (0, 0))],
        out_specs=pl.BlockSpec((RB, D), lambda i: (i, 0)),
        out_shape=jax.ShapeDtypeStruct((N, D), jnp.float32),
    )(p, b.reshape(1, D), W)


def _final_body(p_ref, b_ref, o_ref):
    o_ref[...] = p_ref[0] + p_ref[1] + b_ref[...]


def _final(p, b):
    return pl.pallas_call(
        _final_body,
        grid=(N // RB,),
        in_specs=[pl.BlockSpec((NC, RB, D), lambda i: (0, i, 0)),
                  pl.BlockSpec((1, D), lambda i: (0, 0))],
        out_specs=pl.BlockSpec((RB, D), lambda i: (i, 0)),
        out_shape=jax.ShapeDtypeStruct((N, D), jnp.float32),
    )(p, b.reshape(1, D))


def kernel(x, edge_index, adj_values, W1, b1, W2, b2, W3, b3):
    dst = edge_index[0]
    src = edge_index[1]

    sup1 = _mm_plain(x, W1)
    p1 = _sc_spmv(sup1, src, dst, adj_values)
    sup2 = _mm_fused(p1, b1, W2)
    p2 = _sc_spmv(sup2, src, dst, adj_values)
    sup3 = _mm_fused(p2, b2, W3)
    p3 = _sc_spmv(sup3, src, dst, adj_values)
    return _final(p3, b3)


# speedup vs baseline: 11.1133x; 1.1162x over previous
import functools

import jax
import jax.numpy as jnp
from jax import lax
from jax.experimental import pallas as pl
from jax.experimental.pallas import tpu as pltpu
from jax.experimental.pallas import tpu_sc as plsc

N = 10000
E = 320000
D = 128

NC = 2
NS = 16
NW = NC * NS
EPW = E // NW
C = 80
NCH = EPW // C
RPT = 624
ZR = 16

RB = 1000


NI = 8
NR = 4
UNR = 4
GDEP = 2
IDEP = 4


def _sc_spmv_body(sup_hbm, src_hbm, dst_hbm, val_hbm, out_hbm,
                  acc_sh, zbuf, *bufs):
    IS = list(bufs[0:NI])
    ID = list(bufs[NI:2 * NI])
    VV = list(bufs[2 * NI:3 * NI])
    RW = list(bufs[3 * NI:3 * NI + NR])
    SI = list(bufs[3 * NI + NR:4 * NI + NR])
    SG = list(bufs[4 * NI + NR:4 * NI + 2 * NR])
    SS = list(bufs[4 * NI + 2 * NR:4 * NI + 3 * NR])

    c = lax.axis_index("c")
    s = lax.axis_index("s")
    wid = c * NS + s

    zeros16 = jnp.zeros((16,), jnp.float32)
    for i in range(ZR):
        for j in range(D // 16):
            zbuf[i, pl.ds(j * 16, 16)] = zeros16

    def zero_body(k, carry):
        pltpu.sync_copy(zbuf, acc_sh.at[pl.ds(s * RPT + k * ZR, ZR)])
        return carry

    nz = RPT // ZR + jnp.where(s == NS - 1, (N - NS * RPT) // ZR, 0)
    lax.fori_loop(0, nz, zero_body, 0)
    plsc.subcore_barrier()

    base = wid * EPW

    def idx_start(t, b):
        off = base + t * C
        pltpu.async_copy(src_hbm.at[pl.ds(off, C)], IS[b], SI[b])
        pltpu.async_copy(dst_hbm.at[pl.ds(off, C)], ID[b], SI[b])
        pltpu.async_copy(val_hbm.at[pl.ds(off, C)], VV[b], SI[b])

    def idx_wait(b):
        pltpu.make_async_copy(src_hbm.at[pl.ds(0, C)], IS[b], SI[b]).wait()
        pltpu.make_async_copy(dst_hbm.at[pl.ds(0, C)], ID[b], SI[b]).wait()
        pltpu.make_async_copy(val_hbm.at[pl.ds(0, C)], VV[b], SI[b]).wait()

    def gather_start(bi, br):
        pltpu.async_copy(sup_hbm.at[IS[bi]], RW[br], SG[br])

    def gather_wait(bi, br):
        pltpu.make_async_copy(sup_hbm.at[IS[bi]], RW[br], SG[br]).wait()

    def scatter_start(bi, br):
        pltpu.async_copy(RW[br], acc_sh.at[ID[bi]], SS[br], add=True)

    def scatter_wait(bi, br):
        pltpu.make_async_copy(RW[br], acc_sh.at[ID[bi]], SS[br]).wait()

    def scale(bi, br):
        def ebody(e, iv):
            for u in range(UNR):
                ee = e * UNR + u
                bc = plsc.load_gather(VV[bi], [iv + u])
                for j in range(D // 16):
                    RW[br][ee, pl.ds(j * 16, 16)] = (
                        RW[br][ee, pl.ds(j * 16, 16)] * bc)
            return iv + UNR

        lax.fori_loop(0, C // UNR, ebody, jnp.zeros((16,), jnp.int32))

    def body(t, ph, wait_scatter=True, do_idx=True, do_gather=True):
        bi, br = ph % NI, ph % NR
        if wait_scatter:
            scatter_wait((ph - GDEP) % NI, (ph + GDEP) % NR)
        if do_idx:
            idx_start(t + IDEP, (ph + IDEP) % NI)
        if do_gather:
            idx_wait((ph + GDEP) % NI)
            gather_start((ph + GDEP) % NI, (ph + GDEP) % NR)
        gather_wait(bi, br)
        scale(bi, br)
        scatter_start(bi, br)

    for t in range(IDEP):
        idx_start(t, t)
    for t in range(GDEP):
        idx_wait(t)
        gather_start(t, t)
    for t in range(IDEP):
        body(t, t, wait_scatter=(t >= GDEP))

    NGRP = (NCH - IDEP - IDEP) // NI
    T0 = IDEP

    def group(p, carry):
        t0 = T0 + p * NI
        for u in range(NI):
            body(t0 + u, T0 + u)
        return carry

    lax.fori_loop(0, NGRP, group, 0)

    for t in range(T0 + NGRP * NI, NCH):
        body(t, t,
             do_idx=(t + IDEP <= NCH - 1),
             do_gather=(t + GDEP <= NCH - 1))
    for t in range(NCH - GDEP, NCH):
        scatter_wait(t % NI, t % NR)

    plsc.subcore_barrier()
    pltpu.sync_copy(acc_sh.at[pl.ds(s * RPT, RPT)],
                    out_hbm.at[c, pl.ds(s * RPT, RPT)])

    @pl.when(s == NS - 1)
    def _tail():
        pltpu.sync_copy(acc_sh.at[pl.ds(NS * RPT, N - NS * RPT)],
                        out_hbm.at[c, pl.ds(NS * RPT, N - NS * RPT)])


_sc_spmv = pl.kernel(
    _sc_spmv_body,
    out_type=jax.ShapeDtypeStruct((NC, N, D), jnp.float32),
    mesh=plsc.VectorSubcoreMesh(core_axis_name="c", subcore_axis_name="s",
                                num_cores=NC, num_subcores=NS),
    scratch_types=(
        [pltpu.MemorySpace.VMEM_SHARED((N, D), jnp.float32),
         pltpu.VMEM((ZR, D), jnp.float32)]
        + [pltpu.VMEM((C,), jnp.int32) for _ in range(2 * NI)]
        + [pltpu.VMEM((C,), jnp.float32) for _ in range(NI)]
        + [pltpu.VMEM((C, D), jnp.float32) for _ in range(NR)]
        + [pltpu.SemaphoreType.DMA for _ in range(NI + 2 * NR)]
    ),
    compiler_params=pltpu.CompilerParams(needs_layout_passes=False),
)


def _mm_plain_body(x_ref, w_ref, o_ref):
    o_ref[...] = jnp.dot(x_ref[...], w_ref[...],
                         preferred_element_type=jnp.float32)


def _mm_plain(x, W):
    return pl.pallas_call(
        _mm_plain_body,
        grid=(N // RB,),
        in_specs=[pl.BlockSpec((RB, D), lambda i: (i, 0)),
                  pl.BlockSpec((D, D), lambda i: (0, 0))],
        out_specs=pl.BlockSpec((RB, D), lambda i: (i, 0)),
        out_shape=jax.ShapeDtypeStruct((N, D), jnp.float32),
    )(x, W)


def _mm_fused_body(p_ref, b_ref, w_ref, o_ref):
    h = jnp.maximum(p_ref[0] + p_ref[1] + b_ref[...], 0.0)
    o_ref[...] = jnp.dot(h, w_ref[...], preferred_element_type=jnp.float32)


def _mm_fused(p, b, W):
    return pl.pallas_call(
        _mm_fused_body,
        grid=(N // RB,),
        in_specs=[pl.BlockSpec((NC, RB, D), lambda i: (0, i, 0)),
                  pl.BlockSpec((1, D), lambda i: (0, 0)),
                  pl.BlockSpec((D, D), lambda i: (0, 0))],
        out_specs=pl.BlockSpec((RB, D), lambda i: (i, 0)),
        out_shape=jax.ShapeDtypeStruct((N, D), jnp.float32),
    )(p, b.reshape(1, D), W)


def _final_body(p_ref, b_ref, o_ref):
    o_ref[...] = p_ref[0] + p_ref[1] + b_ref[...]


def _final(p, b):
    return pl.pallas_call(
        _final_body,
        grid=(N // RB,),
        in_specs=[pl.BlockSpec((NC, RB, D), lambda i: (0, i, 0)),
                  pl.BlockSpec((1, D), lambda i: (0, 0))],
        out_specs=pl.BlockSpec((RB, D), lambda i: (i, 0)),
        out_shape=jax.ShapeDtypeStruct((N, D), jnp.float32),
    )(p, b.reshape(1, D))


def kernel(x, edge_index, adj_values, W1, b1, W2, b2, W3, b3):
    dst = edge_index[0]
    src = edge_index[1]

    sup1 = _mm_plain(x, W1)
    p1 = _sc_spmv(sup1, src, dst, adj_values)
    sup2 = _mm_fused(p1, b1, W2)
    p2 = _sc_spmv(sup2, src, dst, adj_values)
    sup3 = _mm_fused(p2, b2, W3)
    p3 = _sc_spmv(sup3, src, dst, adj_values)
    return _final(p3, b3)
